# baseline (speedup 1.0000x reference)
import functools

import jax
import jax.numpy as jnp
from jax import lax
from jax.experimental import pallas as pl
from jax.experimental.pallas import tpu as pltpu
from jax.experimental.pallas import tpu_sc as plsc

_VOCAB = 100000
_D = 64
_NORM_CAP = 10.0
_SEQ = 50
_SEQ_PAD = 56
_NC, _NS = 2, 16
_NW = _NC * _NS
_NSEQ = 22 * 1024
_SEQ_PER_W = _NSEQ // _NW
_G = 8
_CHUNKS = _SEQ_PER_W // _G
_IDX_PER_CHUNK = _G * _SEQ_PAD
_GATHER_SPLIT = 112


def _norm_body(t_ref, o_ref):
    x = t_ref[...]
    ss = jnp.sum(x * x, axis=1, keepdims=True)
    norm = jnp.sqrt(ss)
    scale = jnp.minimum(1.0, _NORM_CAP / jnp.maximum(norm, 1e-7))
    o_ref[...] = x * scale


def _normalize_table(table):
    blk = 2000
    return pl.pallas_call(
        _norm_body,
        grid=(_VOCAB // blk,),
        in_specs=[pl.BlockSpec((blk, _D), lambda i: (i, 0))],
        out_specs=pl.BlockSpec((blk, _D), lambda i: (i, 0)),
        out_shape=jax.ShapeDtypeStruct((_VOCAB, _D), jnp.float32),
    )(table)


_NSPLIT = _IDX_PER_CHUNK // _GATHER_SPLIT
_NB = 2


def _pool_body(table_hbm, idx_hbm, out_hbm, idx_v, rows0, rows1, out0, out1,
               sem0, sem1):
    c = lax.axis_index("c")
    s = lax.axis_index("s")
    wid = s * _NC + c
    rows = (rows0, rows1)
    outs = (out0, out1)
    sems = (sem0, sem1)

    pltpu.sync_copy(
        idx_hbm.at[pl.ds(wid * _SEQ_PER_W * _SEQ_PAD, _SEQ_PER_W * _SEQ_PAD)],
        idx_v)

    def fire(g, b):
        goff = (g % _CHUNKS) * _IDX_PER_CHUNK
        cps = []
        for p in range(_NSPLIT):
            cps.append(pltpu.async_copy(
                table_hbm.at[idx_v.at[pl.ds(goff + p * _GATHER_SPLIT,
                                            _GATHER_SPLIT)]],
                rows[b].at[pl.ds(p * _GATHER_SPLIT, _GATHER_SPLIT)],
                sems[b]))
        return cps

    def drain(b):
        for p in range(_NSPLIT):
            pltpu.make_async_copy(
                table_hbm.at[idx_v.at[pl.ds(p * _GATHER_SPLIT,
                                            _GATHER_SPLIT)]],
                rows[b].at[pl.ds(p * _GATHER_SPLIT, _GATHER_SPLIT)],
                sems[b]).wait()

    fire(0, 0)
    inv = jnp.float32(1.0 / _SEQ)

    def outer(gg, carry):
        for b in range(_NB):
            g = gg * _NB + b
            fire(g + 1, (b + 1) % _NB)
            drain(b)
            for q in range(_G):
                zero = jnp.zeros((16,), jnp.float32)

                def racc(t, acc, b=b, q=q):
                    res = list(acc)
                    for k in range(8):
                        row = q * _SEQ_PAD + t * 8 + k
                        for j in range(4):
                            res[j] = res[j] + rows[b][row, pl.ds(j * 16, 16)]
                    return tuple(res)

                acc = lax.fori_loop(0, _SEQ_PAD // 8, racc, (zero,) * 4)
                for j in range(4):
                    outs[b][pl.ds(q * _D + j * 16, 16)] = acc[j] * inv
            seq_base = wid * _SEQ_PER_W + g * _G
            pltpu.sync_copy(outs[b], out_hbm.at[pl.ds(seq_base * _D, _G * _D)])
        return carry

    lax.fori_loop(0, _CHUNKS // _NB, outer, 0)
    drain(0)


def _pool(table_n, idx_flat):
    mesh = plsc.VectorSubcoreMesh(core_axis_name="c", subcore_axis_name="s")
    fn = pl.kernel(
        _pool_body,
        out_type=jax.ShapeDtypeStruct((_NSEQ * _D,), jnp.float32),
        mesh=mesh,
        scratch_types=[
            pltpu.VMEM((_SEQ_PER_W * _SEQ_PAD,), jnp.int32),
            pltpu.VMEM((_IDX_PER_CHUNK, _D), jnp.float32),
            pltpu.VMEM((_IDX_PER_CHUNK, _D), jnp.float32),
            pltpu.VMEM((_G * _D,), jnp.float32),
            pltpu.VMEM((_G * _D,), jnp.float32),
            pltpu.SemaphoreType.DMA,
            pltpu.SemaphoreType.DMA,
        ],
        compiler_params=pltpu.CompilerParams(use_tc_tiling_on_sc=False),
    )
    return fn(table_n, idx_flat)


def kernel(xs, ys, cands, table):
    xs = xs.astype(jnp.int32)
    ys = ys.astype(jnp.int32)
    cands = cands.astype(jnp.int32)
    n_cands = cands.shape[0]
    idx = jnp.concatenate(
        [xs, ys, cands.reshape(n_cands * cands.shape[1], _SEQ)], axis=0)
    idx = jnp.pad(idx, ((0, 0), (0, _SEQ_PAD - _SEQ)))
    table_n = _normalize_table(table)
    pooled = _pool(table_n, idx.reshape(-1)).reshape(_NSEQ, _D)
    xs_emb = pooled[:1024]
    rest = pooled[1024:]
    xs_enc = jnp.broadcast_to(xs_emb[None], (1 + n_cands, 1024, _D))
    return (xs_enc.reshape(-1, _D), rest)


# speedup vs baseline: 3.8599x; 3.8599x over previous
import jax
import jax.numpy as jnp
from jax import lax
from jax.experimental import pallas as pl
from jax.experimental.pallas import tpu as pltpu
from jax.experimental.pallas import tpu_sc as plsc

_VOCAB = 100000
_D = 64
_NORM_CAP = 10.0
_SEQ = 50
_SEQ_PAD = 56
_NC, _NS = 2, 16
_NW = _NC * _NS
_NSEQ = 22 * 1024
_SEQ_PER_W = _NSEQ // _NW
_G = 4
_GROUPS = _SEQ_PER_W // _G
_IDX_PER_G = _G * _SEQ_PAD
_SPLIT = 112
_NSPLIT = _IDX_PER_G // _SPLIT
_CH = 50000
_NPASS = _VOCAB // _CH
_IDX_W = _SEQ_PER_W * _SEQ_PAD
_OUT_W = _SEQ_PER_W * _D


def _norm_body(t_ref, o_ref):
    x = t_ref[...]
    ss = jnp.sum(x * x, axis=1, keepdims=True)
    norm = jnp.sqrt(ss)
    scale = jnp.minimum(1.0, _NORM_CAP / jnp.maximum(norm, 1e-7))
    o_ref[...] = (x * scale).astype(jnp.bfloat16)


def _normalize_table(table):
    blk = 2000
    return pl.pallas_call(
        _norm_body,
        grid=(_VOCAB // blk,),
        in_specs=[pl.BlockSpec((blk, _D), lambda i: (i, 0))],
        out_specs=pl.BlockSpec((blk, _D), lambda i: (i, 0)),
        out_shape=jax.ShapeDtypeStruct((_VOCAB, _D), jnp.bfloat16),
    )(table)


def _pool_body(table_hbm, zrow_hbm, idx_hbm, out_hbm,
               idx0, idx1, eff0, eff1, rows0, rows1, outg0, outg1, shared,
               sg0, sg1, si0, si1, so0, so1):
    c = lax.axis_index("c")
    s = lax.axis_index("s")
    wid = s * _NC + c
    idxs = (idx0, idx1)
    effs = (eff0, eff1)
    rows = (rows0, rows1)
    outgs = (outg0, outg1)
    sgs = (sg0, sg1)
    sis = (si0, si1)
    sos = (so0, so1)
    ibase = wid * _IDX_W

    def fire_idx(g, b):
        pltpu.async_copy(idx_hbm.at[pl.ds(ibase + g * _IDX_PER_G, _IDX_PER_G)],
                         idxs[b], sis[b])

    def wait_idx(b):
        pltpu.make_async_copy(idx_hbm.at[pl.ds(0, _IDX_PER_G)],
                              idxs[b], sis[b]).wait()

    def compute_eff(b, base):
        def step(i, carry, b=b):
            v = idxs[b][pl.ds(i * 16, 16)]
            ok = (v >= base) & (v < base + _CH)
            effs[b][pl.ds(i * 16, 16)] = jnp.where(ok, v - base, _CH)
            return carry

        lax.fori_loop(0, _IDX_PER_G // 16, step, 0)

    def fire_rows(b):
        for p in range(_NSPLIT):
            pltpu.async_copy(
                shared.at[effs[b].at[pl.ds(p * _SPLIT, _SPLIT)]],
                rows[b].at[pl.ds(p * _SPLIT, _SPLIT)],
                sgs[b])

    def drain_rows(b):
        for p in range(_NSPLIT):
            pltpu.make_async_copy(
                shared.at[effs[b].at[pl.ds(p * _SPLIT, _SPLIT)]],
                rows[b].at[pl.ds(p * _SPLIT, _SPLIT)],
                sgs[b]).wait()

    def drain_out(b):
        pltpu.make_async_copy(outgs[b], out_hbm.at[pl.ds(0, _G * _D)],
                              sos[b]).wait()

    @pl.when(s == 0)
    def _stage_zero():
        pltpu.sync_copy(zrow_hbm, shared.at[pl.ds(_CH, 8), :])

    for vp in range(_NPASS):
        base = vp * _CH
        obase = vp * _NSEQ * _D + wid * _OUT_W
        plsc.subcore_barrier()

        @pl.when(s == 0)
        def _stage(base=base):
            pltpu.sync_copy(table_hbm.at[pl.ds(base, _CH), :],
                            shared.at[pl.ds(0, _CH), :])

        plsc.subcore_barrier()

        fire_idx(0, 0)
        wait_idx(0)
        compute_eff(0, base)
        fire_rows(0)
        fire_idx(1, 1)

        def outer(t, carry, base=base, obase=obase):
            for b in range(2):
                g = t * 2 + b
                nb = 1 - b

                @pl.when(g + 1 < _GROUPS)
                def _next():
                    wait_idx(nb)
                    compute_eff(nb, base)
                    fire_rows(nb)

                @pl.when(g + 2 < _GROUPS)
                def _nexti():
                    fire_idx(g + 2, b)

                drain_rows(b)
                for q in range(_G):
                    acc = (jnp.zeros((16,), jnp.float32),) * 4

                    def racc(r, a, b=b, q=q):
                        a = list(a)
                        for k in range(8):
                            row = q * _SEQ_PAD + r * 8 + k
                            for h in range(2):
                                v = rows[b][row, pl.ds(h * 32, 32)]
                                x, y = plsc.unpack(
                                    v, format=plsc.PackFormat.INTERLEAVED)
                                a[2 * h] = a[2 * h] + x
                                a[2 * h + 1] = a[2 * h + 1] + y
                        return tuple(a)

                    acc = lax.fori_loop(0, _SEQ_PAD // 8, racc, acc)
                    for h in range(2):
                        outgs[b][pl.ds(q * _D + h * 32, 32)] = plsc.pack(
                            acc[2 * h], acc[2 * h + 1],
                            format=plsc.PackFormat.INTERLEAVED)

                @pl.when(g >= 2)
                def _dout():
                    drain_out(b)

                pltpu.async_copy(
                    outgs[b],
                    out_hbm.at[pl.ds(obase + g * _G * _D, _G * _D)],
                    sos[b])
            return carry

        lax.fori_loop(0, _GROUPS // 2, outer, 0)
        drain_out(0)
        drain_out(1)


def _pool(table_n, idx_flat):
    mesh = plsc.VectorSubcoreMesh(core_axis_name="c", subcore_axis_name="s")
    zrow = jnp.zeros((8, _D), jnp.bfloat16)
    fn = pl.kernel(
        _pool_body,
        out_type=jax.ShapeDtypeStruct((_NPASS * _NSEQ * _D,), jnp.bfloat16),
        mesh=mesh,
        scratch_types=[
            pltpu.VMEM((_IDX_PER_G,), jnp.int32),
            pltpu.VMEM((_IDX_PER_G,), jnp.int32),
            pltpu.VMEM((_IDX_PER_G,), jnp.int32),
            pltpu.VMEM((_IDX_PER_G,), jnp.int32),
            pltpu.VMEM((_IDX_PER_G, _D), jnp.bfloat16),
            pltpu.VMEM((_IDX_PER_G, _D), jnp.bfloat16),
            pltpu.VMEM((_G * _D,), jnp.bfloat16),
            pltpu.VMEM((_G * _D,), jnp.bfloat16),
            pltpu.VMEM_SHARED((_CH + 8, _D), jnp.bfloat16),
            pltpu.SemaphoreType.DMA,
            pltpu.SemaphoreType.DMA,
            pltpu.SemaphoreType.DMA,
            pltpu.SemaphoreType.DMA,
            pltpu.SemaphoreType.DMA,
            pltpu.SemaphoreType.DMA,
        ],
        compiler_params=pltpu.CompilerParams(use_tc_tiling_on_sc=False,
                                             needs_layout_passes=False),
    )
    return fn(table_n, zrow, idx_flat)


def kernel(xs, ys, cands, table):
    xs = xs.astype(jnp.int32)
    ys = ys.astype(jnp.int32)
    cands = cands.astype(jnp.int32)
    n_cands = cands.shape[0]
    idx = jnp.concatenate(
        [xs, ys, cands.reshape(n_cands * cands.shape[1], _SEQ)], axis=0)
    idx = jnp.pad(idx, ((0, 0), (0, _SEQ_PAD - _SEQ)))
    table_n = _normalize_table(table)
    partials = _pool(table_n, idx.reshape(-1)).reshape(_NPASS, _NSEQ, _D)
    pooled = (partials[0].astype(jnp.float32)
              + partials[1].astype(jnp.float32)) * (1.0 / _SEQ)
    xs_emb = pooled[:1024]
    rest = pooled[1024:]
    xs_enc = jnp.broadcast_to(xs_emb[None], (1 + n_cands, 1024, _D))
    return (xs_enc.reshape(-1, _D), rest)


# speedup vs baseline: 6.6110x; 1.7127x over previous
import jax
import jax.numpy as jnp
from jax import lax
from jax.experimental import pallas as pl
from jax.experimental.pallas import tpu as pltpu
from jax.experimental.pallas import tpu_sc as plsc

_VOCAB = 100000
_D = 64
_NORM_CAP = 10.0
_SEQ = 50
_SEQ_PAD = 50
_NC, _NS = 2, 16
_NW = _NC * _NS
_NSEQ = 22 * 1024
_SEQ_PER_W = _NSEQ // _NW
_G = 4
_GROUPS = _SEQ_PER_W // _G
_IDX_PER_G = _G * _SEQ_PAD
_SPLITS = (104, 96)
_CH = 50000
_NPASS = _VOCAB // _CH
_IDX_W = _SEQ_PER_W * _SEQ_PAD
_OUT_W = _SEQ_PER_W * _D


def _norm_body(t_ref, o_ref):
    x = t_ref[...]
    ss = jnp.sum(x * x, axis=1, keepdims=True)
    scale = jnp.minimum(1.0, _NORM_CAP * jax.lax.rsqrt(jnp.maximum(ss, 1e-30)))
    o_ref[...] = (x * scale).astype(jnp.bfloat16)


def _normalize_table(table):
    blk = 4000
    return pl.pallas_call(
        _norm_body,
        grid=(_VOCAB // blk,),
        in_specs=[pl.BlockSpec((blk, _D), lambda i: (i, 0))],
        out_specs=pl.BlockSpec((blk, _D), lambda i: (i, 0)),
        out_shape=jax.ShapeDtypeStruct((_VOCAB, _D), jnp.bfloat16),
    )(table)


def _pool_body(table_hbm, zrow_hbm, idx_hbm, out_hbm,
               idx0, idx1, eff0, eff1, rows0, rows1, outg0, outg1, shared,
               sg0, sg1, si0, si1, so0, so1):
    c = lax.axis_index("c")
    s = lax.axis_index("s")
    wid = s * _NC + c
    idxs = (idx0, idx1)
    effs = (eff0, eff1)
    rows = (rows0, rows1)
    outgs = (outg0, outg1)
    sgs = (sg0, sg1)
    sis = (si0, si1)
    sos = (so0, so1)
    ibase = wid * _IDX_W

    def fire_idx(g, b):
        pltpu.async_copy(idx_hbm.at[pl.ds(ibase + g * _IDX_PER_G, _IDX_PER_G)],
                         idxs[b], sis[b])

    def wait_idx(b):
        pltpu.make_async_copy(idx_hbm.at[pl.ds(0, _IDX_PER_G)],
                              idxs[b], sis[b]).wait()

    def compute_eff(b, base):
        def eff16(o, b=b):
            v = idxs[b][pl.ds(o, 16)]
            ok = (v >= base) & (v < base + _CH)
            effs[b][pl.ds(o, 16)] = jnp.where(ok, v - base,
                                              _CH + (v & 7))

        def step(i, carry):
            eff16(i * 16)
            return carry

        lax.fori_loop(0, _IDX_PER_G // 16, step, 0)
        if _IDX_PER_G % 16:
            eff16(_IDX_PER_G - 16)

    def fire_rows(b):
        off = 0
        for n in _SPLITS:
            pltpu.async_copy(
                shared.at[effs[b].at[pl.ds(off, n)]],
                rows[b].at[pl.ds(off, n)],
                sgs[b])
            off += n

    def drain_rows(b):
        off = 0
        for n in _SPLITS:
            pltpu.make_async_copy(
                shared.at[effs[b].at[pl.ds(off, n)]],
                rows[b].at[pl.ds(off, n)],
                sgs[b]).wait()
            off += n

    def drain_out(b):
        pltpu.make_async_copy(outgs[b], out_hbm.at[pl.ds(0, _G * _D)],
                              sos[b]).wait()

    @pl.when(s == 0)
    def _stage_zero():
        pltpu.sync_copy(zrow_hbm, shared.at[pl.ds(_CH, 8), :])

    for vp in range(_NPASS):
        base = vp * _CH
        obase = vp * _NSEQ * _D + wid * _OUT_W
        plsc.subcore_barrier()

        @pl.when(s == 0)
        def _stage(base=base):
            pltpu.sync_copy(table_hbm.at[pl.ds(base, _CH), :],
                            shared.at[pl.ds(0, _CH), :])

        plsc.subcore_barrier()

        fire_idx(0, 0)
        wait_idx(0)
        compute_eff(0, base)
        fire_rows(0)
        fire_idx(1, 1)

        def outer(t, carry, base=base, obase=obase):
            for b in range(2):
                g = t * 2 + b
                nb = 1 - b

                @pl.when(g + 1 < _GROUPS)
                def _next():
                    wait_idx(nb)
                    compute_eff(nb, base)
                    fire_rows(nb)

                @pl.when(g + 2 < _GROUPS)
                def _nexti():
                    fire_idx(g + 2, b)

                drain_rows(b)
                for q in range(_G):
                    acc = (jnp.zeros((16,), jnp.float32),) * 4

                    def racc(r, a, b=b, q=q, nrows=8):
                        a = list(a)
                        for k in range(nrows):
                            row = q * _SEQ_PAD + r * 8 + k
                            for h in range(2):
                                v = rows[b][row, pl.ds(h * 32, 32)]
                                x, y = plsc.unpack(
                                    v, format=plsc.PackFormat.INTERLEAVED)
                                a[2 * h] = a[2 * h] + x
                                a[2 * h + 1] = a[2 * h + 1] + y
                        return tuple(a)

                    acc = lax.fori_loop(0, _SEQ_PAD // 8, racc, acc)
                    acc = racc(jnp.int32(_SEQ_PAD // 8), acc, nrows=2)
                    for h in range(2):
                        outgs[b][pl.ds(q * _D + h * 32, 32)] = plsc.pack(
                            acc[2 * h], acc[2 * h + 1],
                            format=plsc.PackFormat.INTERLEAVED)

                @pl.when(g >= 2)
                def _dout():
                    drain_out(b)

                pltpu.async_copy(
                    outgs[b],
                    out_hbm.at[pl.ds(obase + g * _G * _D, _G * _D)],
                    sos[b])
            return carry

        lax.fori_loop(0, _GROUPS // 2, outer, 0)
        drain_out(0)
        drain_out(1)


def _pool(table_n, idx_flat):
    mesh = plsc.VectorSubcoreMesh(core_axis_name="c", subcore_axis_name="s")
    zrow = jnp.zeros((8, _D), jnp.bfloat16)
    fn = pl.kernel(
        _pool_body,
        out_type=jax.ShapeDtypeStruct((_NPASS * _NSEQ * _D,), jnp.bfloat16),
        mesh=mesh,
        scratch_types=[
            pltpu.VMEM((_IDX_PER_G,), jnp.int32),
            pltpu.VMEM((_IDX_PER_G,), jnp.int32),
            pltpu.VMEM((_IDX_PER_G,), jnp.int32),
            pltpu.VMEM((_IDX_PER_G,), jnp.int32),
            pltpu.VMEM((_IDX_PER_G, _D), jnp.bfloat16),
            pltpu.VMEM((_IDX_PER_G, _D), jnp.bfloat16),
            pltpu.VMEM((_G * _D,), jnp.bfloat16),
            pltpu.VMEM((_G * _D,), jnp.bfloat16),
            pltpu.VMEM_SHARED((_CH + 8, _D), jnp.bfloat16),
            pltpu.SemaphoreType.DMA,
            pltpu.SemaphoreType.DMA,
            pltpu.SemaphoreType.DMA,
            pltpu.SemaphoreType.DMA,
            pltpu.SemaphoreType.DMA,
            pltpu.SemaphoreType.DMA,
        ],
        compiler_params=pltpu.CompilerParams(use_tc_tiling_on_sc=False,
                                             needs_layout_passes=False),
    )
    return fn(table_n, zrow, idx_flat)


def kernel(xs, ys, cands, table):
    xs = xs.astype(jnp.int32)
    ys = ys.astype(jnp.int32)
    cands = cands.astype(jnp.int32)
    n_cands = cands.shape[0]
    idx = jnp.concatenate(
        [xs, ys, cands.reshape(n_cands * cands.shape[1], _SEQ)], axis=0)
    table_n = _normalize_table(table)
    partials = _pool(table_n, idx.reshape(-1)).reshape(_NPASS, _NSEQ, _D)
    pooled = (partials[0].astype(jnp.float32)
              + partials[1].astype(jnp.float32)) * (1.0 / _SEQ)
    xs_emb = pooled[:1024]
    rest = pooled[1024:]
    xs_enc = jnp.broadcast_to(xs_emb[None], (1 + n_cands, 1024, _D))
    return (xs_enc.reshape(-1, _D), rest)
